# baseline (speedup 1.0000x reference)
import functools

import jax
import jax.numpy as jnp
from jax import lax
from jax.experimental import pallas as pl
from jax.experimental.pallas import tpu as pltpu
from jax.experimental.pallas import tpu_sc as plsc

B, S, D = 2, 2048, 1024
E, FF = 64, 1024
N = B * S
CAP = 2 * ((N + E - 1) // E)
SENTINEL = E * CAP

TB = 512
NC, NS = 2, 16
NW = NC * NS
SLOTS_PER_W = E * CAP // NW
TOK_PER_W = N // NW
GCHUNK = 32


def _routing_body(x_ref, gw_ref, logits_ref, idx_ref, cnt_ref, base_ref):
    t = pl.program_id(0)

    @pl.when(t == 0)
    def _():
        base_ref[...] = jnp.zeros_like(base_ref)

    x = x_ref[...]
    logits = lax.dot_general(x, gw_ref[...], (((1,), (1,)), ((), ())),
                             preferred_element_type=jnp.float32)
    logits_ref[...] = logits
    m = jnp.max(logits, axis=1, keepdims=True)
    eidx = lax.broadcasted_iota(jnp.int32, (TB, E), 1)
    sel = jnp.min(jnp.where(logits == m, eidx, E), axis=1)
    oh = (eidx == sel[:, None]).astype(jnp.float32)
    r = lax.broadcasted_iota(jnp.int32, (TB, TB), 0)
    c = lax.broadcasted_iota(jnp.int32, (TB, TB), 1)
    ltri = (c <= r).astype(jnp.float32)
    csum = lax.dot_general(ltri, oh, (((1,), (0,)), ((), ())),
                           preferred_element_type=jnp.float32)
    csum = csum + base_ref[...]
    base_ref[...] = csum[TB - 1:TB, :]
    cnt_ref[0, 0, :] = csum[TB - 1, :]
    pos = jnp.sum(csum * oh, axis=1).astype(jnp.int32) - 1
    slot = sel * CAP + pos
    idx_ref[0, 0, :] = jnp.where(pos < CAP, slot, SENTINEL)


def _routing(x, gate_w):
    return pl.pallas_call(
        _routing_body,
        grid=(N // TB,),
        in_specs=[
            pl.BlockSpec((TB, D), lambda t: (t, 0)),
            pl.BlockSpec((E, D), lambda t: (0, 0)),
        ],
        out_specs=[
            pl.BlockSpec((TB, E), lambda t: (t, 0)),
            pl.BlockSpec((1, 1, TB), lambda t: (t, 0, 0)),
            pl.BlockSpec((1, 1, E), lambda t: (0, 0, 0)),
        ],
        out_shape=[
            jax.ShapeDtypeStruct((N, E), jnp.float32),
            jax.ShapeDtypeStruct((N // TB, 1, TB), jnp.int32),
            jax.ShapeDtypeStruct((1, 1, E), jnp.float32),
        ],
        scratch_shapes=[pltpu.VMEM((1, E), jnp.float32)],
    )(x, gate_w)


NCHUNK = SLOTS_PER_W // GCHUNK


def _dispatch_body(idx_hbm, x_hbm, buf_hbm, idx_v, tid_v, rows_v,
                   gsem, wsem):
    wid = lax.axis_index("s") * NC + lax.axis_index("c")
    base_slot = pl.multiple_of(wid * SLOTS_PER_W, SLOTS_PER_W)
    pltpu.sync_copy(idx_hbm, idx_v)
    iota16 = lax.iota(jnp.int32, 16)
    neg16 = jnp.full((16,), -1, jnp.int32)
    for ci in range(NCHUNK):
        for i in range(GCHUNK // 16):
            tid_v[ci, pl.ds(i * 16, 16)] = neg16

    def body(j, carry):
        sl = idx_v[pl.ds(j * 16, 16)]
        rel = sl - base_slot
        msk = (rel >= 0) & (rel < SLOTS_PER_W)
        relc = jnp.clip(rel, 0, SLOTS_PER_W - 1)
        tok = lax.iota(jnp.int32, 16) + j * 16
        plsc.store_scatter(tid_v, [relc // GCHUNK, relc % GCHUNK], tok,
                           mask=msk)
        return carry

    lax.fori_loop(0, N // 16, body, 0)
    filled = []
    for ci in range(NCHUNK):
        filled.append(jnp.max(tid_v[ci, pl.ds(0, 16)]) >= 0)
        for i in range(GCHUNK // 16):
            v = tid_v[ci, pl.ds(i * 16, 16)]
            fillv = iota16 + ((base_slot + ci * GCHUNK + i * 16) & (N - 1))
            tid_v[ci, pl.ds(i * 16, 16)] = jnp.where(v < 0, fillv, v)
    gd = [pltpu.make_async_copy(x_hbm.at[tid_v.at[ci]],
                                rows_v.at[ci % 2], gsem)
          for ci in range(NCHUNK)]
    wd = [pltpu.make_async_copy(rows_v.at[ci % 2],
                                buf_hbm.at[wid * NCHUNK + ci], wsem)
          for ci in range(NCHUNK)]

    def when_do(cond, *fns):
        @pl.when(cond)
        def _():
            for f in fns:
                f()

    for ci in range(NCHUNK):
        if ci >= 2:
            when_do(filled[ci - 2], wd[ci - 2].wait)
        when_do(filled[ci], gd[ci].start)
        if ci >= 1:
            when_do(filled[ci - 1], gd[ci - 1].wait, wd[ci - 1].start)
    last = NCHUNK - 1
    when_do(filled[last], gd[last].wait, wd[last].start)
    if last >= 1:
        when_do(filled[last - 1], wd[last - 1].wait)
    when_do(filled[last], wd[last].wait)


@functools.cache
def _dispatch():
    return pl.kernel(
        _dispatch_body,
        out_type=jax.ShapeDtypeStruct((E * CAP // GCHUNK, GCHUNK, D),
                                      jnp.float32),
        mesh=plsc.VectorSubcoreMesh(core_axis_name="c", subcore_axis_name="s"),
        scratch_types=[
            pltpu.VMEM((N,), jnp.int32),
            pltpu.VMEM((NCHUNK, GCHUNK), jnp.int32),
            pltpu.VMEM((2, GCHUNK, D), jnp.float32),
            pltpu.SemaphoreType.DMA,
            pltpu.SemaphoreType.DMA,
        ],
        compiler_params=pltpu.CompilerParams(needs_layout_passes=False),
    )


FROWS = 32
CT = CAP // FROWS


def _ffn_body(cnt_ref, buf_ref, w1_ref, w3_ref, w2_ref, ob_ref):
    e = pl.program_id(0)
    rt = pl.program_id(1)
    valid = rt * FROWS < cnt_ref[jnp.minimum(e, E - 1)]

    @pl.when((e < E) & valid)
    def _():
        x = buf_ref[...]
        a = jnp.dot(x, w1_ref[0], preferred_element_type=jnp.float32)
        b = jnp.dot(x, w3_ref[0], preferred_element_type=jnp.float32)
        g = a * (1.0 / (1.0 + jnp.exp(-a))) * b
        ob_ref[...] = jnp.dot(g, w2_ref[0], preferred_element_type=jnp.float32)

    @pl.when(e == E)
    def _():
        ob_ref[...] = jnp.zeros_like(ob_ref)


def _ffn(counts, buf, w1, w3, w2):
    wmap = lambda e, rt, c: (jnp.minimum(e, E - 1), 0, 0)

    def lastv(c, ee):
        return jnp.clip((c[ee] + FROWS - 1) // FROWS - 1, 0, CT - 1)

    def bufmap(e, rt, c):
        ee = jnp.minimum(e, E - 1)
        return (ee * CT + jnp.where(rt * FROWS < c[ee], rt, lastv(c, ee)), 0)

    def obmap(e, rt, c):
        ee = jnp.minimum(e, E - 1)
        keep = ee * CT + jnp.where(rt * FROWS < c[ee], rt, lastv(c, ee))
        return (jnp.where(e == E, E * CT + rt, keep), 0)

    return pl.pallas_call(
        _ffn_body,
        grid_spec=pltpu.PrefetchScalarGridSpec(
            num_scalar_prefetch=1,
            grid=(E + 1, CT),
            in_specs=[
                pl.BlockSpec((FROWS, D), bufmap),
                pl.BlockSpec((1, D, FF), wmap),
                pl.BlockSpec((1, D, FF), wmap),
                pl.BlockSpec((1, FF, D), wmap),
            ],
            out_specs=pl.BlockSpec((FROWS, D), obmap),
        ),
        out_shape=jax.ShapeDtypeStruct((E * CAP + CAP, D), jnp.float32),
    )(counts, buf, w1, w3, w2)


CCHUNK = TOK_PER_W // GCHUNK


def _combine_body(idx_hbm, ob_hbm, out_hbm, idx_v, rows_v, gsem, wsem):
    wid = lax.axis_index("s") * NC + lax.axis_index("c")
    base = wid * TOK_PER_W
    pltpu.sync_copy(idx_hbm.at[pl.ds(wid * CCHUNK, CCHUNK)], idx_v)
    gathers = [None, None]
    writes = [None, None]
    for ci in range(CCHUNK):
        b = ci % 2
        if writes[b] is not None:
            writes[b].wait()
        gathers[b] = pltpu.async_copy(ob_hbm.at[idx_v.at[ci]],
                                      rows_v.at[b], gsem)
        if ci >= 1:
            pb = (ci - 1) % 2
            gathers[pb].wait()
            writes[pb] = pltpu.async_copy(
                rows_v.at[pb],
                out_hbm.at[pl.ds(base + (ci - 1) * GCHUNK, GCHUNK), :],
                wsem)
    last = CCHUNK - 1
    gathers[last % 2].wait()
    pltpu.sync_copy(rows_v.at[last % 2],
                    out_hbm.at[pl.ds(base + last * GCHUNK, GCHUNK), :])
    if writes[(last - 1) % 2] is not None:
        writes[(last - 1) % 2].wait()


@functools.cache
def _combine():
    return pl.kernel(
        _combine_body,
        out_type=jax.ShapeDtypeStruct((N, D), jnp.float32),
        mesh=plsc.VectorSubcoreMesh(core_axis_name="c", subcore_axis_name="s"),
        scratch_types=[
            pltpu.VMEM((CCHUNK, GCHUNK), jnp.int32),
            pltpu.VMEM((2, GCHUNK, D), jnp.float32),
            pltpu.SemaphoreType.DMA,
            pltpu.SemaphoreType.DMA,
        ],
    )


def kernel(hidden_states, gate_w, w1, w2, w3):
    x = hidden_states.reshape(N, D)
    logits, idx3, cnt3 = _routing(x, gate_w)
    idx = idx3.reshape(N)
    counts = cnt3.reshape(E).astype(jnp.int32)
    buf = _dispatch()(idx, x).reshape(E * CAP, D)
    ob = _ffn(counts, buf, w1, w3, w2)
    out = _combine()(idx.reshape(N // GCHUNK, GCHUNK), ob)
    return out.reshape(B, S, D), logits


# speedup vs baseline: 1.8267x; 1.8267x over previous
import functools

import jax
import jax.numpy as jnp
from jax import lax
from jax.experimental import pallas as pl
from jax.experimental.pallas import tpu as pltpu
from jax.experimental.pallas import tpu_sc as plsc

B, S, D = 2, 2048, 1024
E, FF = 64, 1024
N = B * S
CAP = 2 * ((N + E - 1) // E)
SENTINEL = E * CAP

TB = 512
NC, NS = 2, 16
NW = NC * NS
SLOTS_PER_W = E * CAP // NW
TOK_PER_W = N // NW
GCHUNK = 32


def _routing_body(x_ref, gw_ref, logits_ref, idx_ref, base_ref):
    t = pl.program_id(0)

    @pl.when(t == 0)
    def _():
        base_ref[...] = jnp.zeros_like(base_ref)

    x = x_ref[...]
    logits = lax.dot_general(x, gw_ref[...], (((1,), (1,)), ((), ())),
                             preferred_element_type=jnp.float32)
    logits_ref[...] = logits
    m = jnp.max(logits, axis=1, keepdims=True)
    eidx = lax.broadcasted_iota(jnp.int32, (TB, E), 1)
    sel = jnp.min(jnp.where(logits == m, eidx, E), axis=1)
    oh = (eidx == sel[:, None]).astype(jnp.float32)
    r = lax.broadcasted_iota(jnp.int32, (TB, TB), 0)
    c = lax.broadcasted_iota(jnp.int32, (TB, TB), 1)
    ltri = (c <= r).astype(jnp.float32)
    csum = lax.dot_general(ltri, oh, (((1,), (0,)), ((), ())),
                           preferred_element_type=jnp.float32)
    csum = csum + base_ref[...]
    base_ref[...] = csum[TB - 1:TB, :]
    pos = jnp.sum(csum * oh, axis=1).astype(jnp.int32) - 1
    slot = sel * CAP + pos
    idx_ref[0, 0, :] = jnp.where(pos < CAP, slot, SENTINEL)


def _routing(x, gate_w):
    return pl.pallas_call(
        _routing_body,
        grid=(N // TB,),
        in_specs=[
            pl.BlockSpec((TB, D), lambda t: (t, 0)),
            pl.BlockSpec((E, D), lambda t: (0, 0)),
        ],
        out_specs=[
            pl.BlockSpec((TB, E), lambda t: (t, 0)),
            pl.BlockSpec((1, 1, TB), lambda t: (t, 0, 0)),
        ],
        out_shape=[
            jax.ShapeDtypeStruct((N, E), jnp.float32),
            jax.ShapeDtypeStruct((N // TB, 1, TB), jnp.int32),
        ],
        scratch_shapes=[pltpu.VMEM((1, E), jnp.float32)],
    )(x, gate_w)


NCHUNK = SLOTS_PER_W // GCHUNK


def _dispatch_body(idx_hbm, x_hbm, buf_hbm, idx_v, tid_v, rows_v,
                   gsem, wsem):
    wid = lax.axis_index("s") * NC + lax.axis_index("c")
    base_slot = pl.multiple_of(wid * SLOTS_PER_W, SLOTS_PER_W)
    pltpu.sync_copy(idx_hbm, idx_v)
    iota16 = lax.iota(jnp.int32, 16)
    neg16 = jnp.full((16,), -1, jnp.int32)
    for ci in range(NCHUNK):
        for i in range(GCHUNK // 16):
            tid_v[ci, pl.ds(i * 16, 16)] = neg16

    def body(j, carry):
        sl = idx_v[pl.ds(j * 16, 16)]
        rel = sl - base_slot
        msk = (rel >= 0) & (rel < SLOTS_PER_W)
        relc = jnp.clip(rel, 0, SLOTS_PER_W - 1)
        tok = lax.iota(jnp.int32, 16) + j * 16
        plsc.store_scatter(tid_v, [relc // GCHUNK, relc % GCHUNK], tok,
                           mask=msk)
        return carry

    lax.fori_loop(0, N // 16, body, 0)
    filled = []
    for ci in range(NCHUNK):
        filled.append(jnp.max(tid_v[ci, pl.ds(0, 16)]) >= 0)
        for i in range(GCHUNK // 16):
            v = tid_v[ci, pl.ds(i * 16, 16)]
            fillv = iota16 + ((base_slot + ci * GCHUNK + i * 16) & (N - 1))
            tid_v[ci, pl.ds(i * 16, 16)] = jnp.where(v < 0, fillv, v)
    gd = [pltpu.make_async_copy(x_hbm.at[tid_v.at[ci]],
                                rows_v.at[ci % 2], gsem)
          for ci in range(NCHUNK)]
    wd = [pltpu.make_async_copy(rows_v.at[ci % 2],
                                buf_hbm.at[wid * NCHUNK + ci], wsem)
          for ci in range(NCHUNK)]

    def when_do(cond, *fns):
        @pl.when(cond)
        def _():
            for f in fns:
                f()

    for ci in range(NCHUNK):
        if ci >= 2:
            when_do(filled[ci - 2], wd[ci - 2].wait)
        when_do(filled[ci], gd[ci].start)
        if ci >= 1:
            when_do(filled[ci - 1], gd[ci - 1].wait, wd[ci - 1].start)
    last = NCHUNK - 1
    when_do(filled[last], gd[last].wait, wd[last].start)
    if last >= 1:
        when_do(filled[last - 1], wd[last - 1].wait)
    when_do(filled[last], wd[last].wait)


@functools.cache
def _dispatch():
    return pl.kernel(
        _dispatch_body,
        out_type=jax.ShapeDtypeStruct((E * CAP // GCHUNK, GCHUNK, D),
                                      jnp.float32),
        mesh=plsc.VectorSubcoreMesh(core_axis_name="c", subcore_axis_name="s"),
        scratch_types=[
            pltpu.VMEM((N,), jnp.int32),
            pltpu.VMEM((NCHUNK, GCHUNK), jnp.int32),
            pltpu.VMEM((2, GCHUNK, D), jnp.float32),
            pltpu.SemaphoreType.DMA,
            pltpu.SemaphoreType.DMA,
        ],
        compiler_params=pltpu.CompilerParams(needs_layout_passes=False),
    )


def _ffn_body(buf_ref, w1_ref, w3_ref, w2_ref, ob_ref):
    e = pl.program_id(0)

    @pl.when(e < E)
    def _():
        x = buf_ref[...]
        a = jnp.dot(x, w1_ref[0], preferred_element_type=jnp.float32)
        b = jnp.dot(x, w3_ref[0], preferred_element_type=jnp.float32)
        g = a * (1.0 / (1.0 + jnp.exp(-a))) * b
        ob_ref[...] = jnp.dot(g, w2_ref[0], preferred_element_type=jnp.float32)

    @pl.when(e == E)
    def _():
        ob_ref[...] = jnp.zeros_like(ob_ref)


def _ffn(buf, w1, w3, w2):
    wmap = lambda e: (jnp.minimum(e, E - 1), 0, 0)
    return pl.pallas_call(
        _ffn_body,
        grid=(E + 1,),
        in_specs=[
            pl.BlockSpec((CAP, D), lambda e: (jnp.minimum(e, E - 1), 0)),
            pl.BlockSpec((1, D, FF), wmap),
            pl.BlockSpec((1, D, FF), wmap),
            pl.BlockSpec((1, FF, D), wmap),
        ],
        out_specs=pl.BlockSpec((CAP, D), lambda e: (e, 0)),
        out_shape=jax.ShapeDtypeStruct((E * CAP + CAP, D), jnp.float32),
    )(buf, w1, w3, w2)


CCHUNK = TOK_PER_W // GCHUNK


def _combine_body(idx_hbm, ob_hbm, out_hbm, idx_v, rows_v, gsem, wsem):
    wid = lax.axis_index("s") * NC + lax.axis_index("c")
    base = wid * TOK_PER_W
    pltpu.sync_copy(idx_hbm.at[pl.ds(wid * CCHUNK, CCHUNK)], idx_v)
    gathers = [None, None]
    writes = [None, None]
    for ci in range(CCHUNK):
        b = ci % 2
        if writes[b] is not None:
            writes[b].wait()
        gathers[b] = pltpu.async_copy(ob_hbm.at[idx_v.at[ci]],
                                      rows_v.at[b], gsem)
        if ci >= 1:
            pb = (ci - 1) % 2
            gathers[pb].wait()
            writes[pb] = pltpu.async_copy(
                rows_v.at[pb],
                out_hbm.at[pl.ds(base + (ci - 1) * GCHUNK, GCHUNK), :],
                wsem)
    last = CCHUNK - 1
    gathers[last % 2].wait()
    pltpu.sync_copy(rows_v.at[last % 2],
                    out_hbm.at[pl.ds(base + last * GCHUNK, GCHUNK), :])
    if writes[(last - 1) % 2] is not None:
        writes[(last - 1) % 2].wait()


@functools.cache
def _combine():
    return pl.kernel(
        _combine_body,
        out_type=jax.ShapeDtypeStruct((N, D), jnp.float32),
        mesh=plsc.VectorSubcoreMesh(core_axis_name="c", subcore_axis_name="s"),
        scratch_types=[
            pltpu.VMEM((CCHUNK, GCHUNK), jnp.int32),
            pltpu.VMEM((2, GCHUNK, D), jnp.float32),
            pltpu.SemaphoreType.DMA,
            pltpu.SemaphoreType.DMA,
        ],
    )


def kernel(hidden_states, gate_w, w1, w2, w3):
    x = hidden_states.reshape(N, D)
    logits, idx3 = _routing(x, gate_w)
    idx = idx3.reshape(N)
    buf = _dispatch()(idx, x).reshape(E * CAP, D)
    ob = _ffn(buf, w1, w3, w2)
    out = _combine()(idx.reshape(N // GCHUNK, GCHUNK), ob)
    return out.reshape(B, S, D), logits


# speedup vs baseline: 1.8284x; 1.0009x over previous
import functools

import jax
import jax.numpy as jnp
from jax import lax
from jax.experimental import pallas as pl
from jax.experimental.pallas import tpu as pltpu
from jax.experimental.pallas import tpu_sc as plsc

B, S, D = 2, 2048, 1024
E, FF = 64, 1024
N = B * S
CAP = 2 * ((N + E - 1) // E)
SENTINEL = E * CAP

TB = 512
NC, NS = 2, 16
NW = NC * NS
SLOTS_PER_W = E * CAP // NW
TOK_PER_W = N // NW
GCHUNK = 32
NBUF = 3


def _routing_body(x_ref, gw_ref, logits_ref, idx_ref, base_ref):
    t = pl.program_id(0)

    @pl.when(t == 0)
    def _():
        base_ref[...] = jnp.zeros_like(base_ref)

    x = x_ref[...]
    logits = lax.dot_general(x, gw_ref[...], (((1,), (1,)), ((), ())),
                             preferred_element_type=jnp.float32)
    logits_ref[...] = logits
    m = jnp.max(logits, axis=1, keepdims=True)
    eidx = lax.broadcasted_iota(jnp.int32, (TB, E), 1)
    sel = jnp.min(jnp.where(logits == m, eidx, E), axis=1)
    oh = (eidx == sel[:, None]).astype(jnp.float32)
    r = lax.broadcasted_iota(jnp.int32, (TB, TB), 0)
    c = lax.broadcasted_iota(jnp.int32, (TB, TB), 1)
    ltri = (c <= r).astype(jnp.float32)
    csum = lax.dot_general(ltri, oh, (((1,), (0,)), ((), ())),
                           preferred_element_type=jnp.float32)
    csum = csum + base_ref[...]
    base_ref[...] = csum[TB - 1:TB, :]
    pos = jnp.sum(csum * oh, axis=1).astype(jnp.int32) - 1
    slot = sel * CAP + pos
    idx_ref[...] = jnp.where(pos < CAP, slot, SENTINEL)


def _routing(x, gate_w):
    return pl.pallas_call(
        _routing_body,
        grid=(N // TB,),
        in_specs=[
            pl.BlockSpec((TB, D), lambda t: (t, 0)),
            pl.BlockSpec((E, D), lambda t: (0, 0)),
        ],
        out_specs=[
            pl.BlockSpec((TB, E), lambda t: (t, 0)),
            pl.BlockSpec((TB,), lambda t: (t,)),
        ],
        out_shape=[
            jax.ShapeDtypeStruct((N, E), jnp.float32),
            jax.ShapeDtypeStruct((N,), jnp.int32),
        ],
        scratch_shapes=[pltpu.VMEM((1, E), jnp.float32)],
    )(x, gate_w)


NCHUNK = SLOTS_PER_W // GCHUNK


def _dispatch_body(idx_hbm, x_hbm, buf_hbm, idx_v, tid_v, rows_v,
                   gsem, wsem):
    wid = lax.axis_index("s") * NC + lax.axis_index("c")
    base_slot = pl.multiple_of(wid * SLOTS_PER_W, SLOTS_PER_W)
    pltpu.sync_copy(idx_hbm, idx_v)
    iota16 = lax.iota(jnp.int32, 16)
    neg16 = jnp.full((16,), -1, jnp.int32)
    for ci in range(NCHUNK):
        for i in range(GCHUNK // 16):
            tid_v[ci, pl.ds(i * 16, 16)] = neg16

    def body(j, carry):
        sl = idx_v[pl.ds(j * 16, 16)]
        rel = sl - base_slot
        msk = (rel >= 0) & (rel < SLOTS_PER_W)
        relc = jnp.clip(rel, 0, SLOTS_PER_W - 1)
        tok = lax.iota(jnp.int32, 16) + j * 16
        plsc.store_scatter(tid_v, [relc // GCHUNK, relc % GCHUNK], tok,
                           mask=msk)
        return carry

    lax.fori_loop(0, N // 16, body, 0)
    filled = []
    for ci in range(NCHUNK):
        filled.append(jnp.max(tid_v[ci, pl.ds(0, 16)]) >= 0)
        for i in range(GCHUNK // 16):
            v = tid_v[ci, pl.ds(i * 16, 16)]
            fillv = iota16 + ((base_slot + ci * GCHUNK + i * 16) & (N - 1))
            tid_v[ci, pl.ds(i * 16, 16)] = jnp.where(v < 0, fillv, v)
    gd = [pltpu.make_async_copy(x_hbm.at[tid_v.at[ci]],
                                rows_v.at[ci % NBUF], gsem)
          for ci in range(NCHUNK)]
    wd = [pltpu.make_async_copy(rows_v.at[ci % NBUF],
                                buf_hbm.at[wid * NCHUNK + ci], wsem)
          for ci in range(NCHUNK)]

    def when_do(cond, *fns):
        @pl.when(cond)
        def _():
            for f in fns:
                f()

    for ci in range(NCHUNK):
        if ci >= NBUF:
            when_do(filled[ci - NBUF], wd[ci - NBUF].wait)
        when_do(filled[ci], gd[ci].start)
        if ci >= 1:
            when_do(filled[ci - 1], gd[ci - 1].wait, wd[ci - 1].start)
    last = NCHUNK - 1
    when_do(filled[last], gd[last].wait, wd[last].start)
    for ci in range(max(0, NCHUNK - NBUF + 1), NCHUNK):
        when_do(filled[ci], wd[ci].wait)


@functools.cache
def _dispatch():
    return pl.kernel(
        _dispatch_body,
        out_type=jax.ShapeDtypeStruct((E * CAP // GCHUNK, GCHUNK, D),
                                      jnp.float32),
        mesh=plsc.VectorSubcoreMesh(core_axis_name="c", subcore_axis_name="s"),
        scratch_types=[
            pltpu.VMEM((N,), jnp.int32),
            pltpu.VMEM((NCHUNK, GCHUNK), jnp.int32),
            pltpu.VMEM((NBUF, GCHUNK, D), jnp.float32),
            pltpu.SemaphoreType.DMA,
            pltpu.SemaphoreType.DMA,
        ],
        compiler_params=pltpu.CompilerParams(needs_layout_passes=False),
    )


def _ffn_body(buf_ref, w1_ref, w3_ref, w2_ref, ob_ref):
    e = pl.program_id(0)

    @pl.when(e < E)
    def _():
        x = buf_ref[...]
        a = jnp.dot(x, w1_ref[0], preferred_element_type=jnp.float32)
        b = jnp.dot(x, w3_ref[0], preferred_element_type=jnp.float32)
        g = a * (1.0 / (1.0 + jnp.exp(-a))) * b
        ob_ref[...] = jnp.dot(g, w2_ref[0], preferred_element_type=jnp.float32)

    @pl.when(e == E)
    def _():
        ob_ref[...] = jnp.zeros_like(ob_ref)


def _ffn(buf, w1, w3, w2):
    wmap = lambda e: (jnp.minimum(e, E - 1), 0, 0)
    return pl.pallas_call(
        _ffn_body,
        grid=(E + 1,),
        in_specs=[
            pl.BlockSpec((CAP, D), lambda e: (jnp.minimum(e, E - 1), 0)),
            pl.BlockSpec((1, D, FF), wmap),
            pl.BlockSpec((1, D, FF), wmap),
            pl.BlockSpec((1, FF, D), wmap),
        ],
        out_specs=pl.BlockSpec((CAP, D), lambda e: (e, 0)),
        out_shape=jax.ShapeDtypeStruct((E * CAP + CAP, D), jnp.float32),
    )(buf, w1, w3, w2)


CCHUNK = TOK_PER_W // GCHUNK


def _combine_body(idx_hbm, ob_hbm, out_hbm, idx_v, rows_v, gsem, wsem):
    wid = lax.axis_index("s") * NC + lax.axis_index("c")
    base = pl.multiple_of(wid * TOK_PER_W, TOK_PER_W)
    pltpu.sync_copy(idx_hbm.at[pl.ds(base, TOK_PER_W)], idx_v)
    gd = [pltpu.make_async_copy(
              ob_hbm.at[idx_v.at[pl.ds(ci * GCHUNK, GCHUNK)]],
              rows_v.at[ci % NBUF], gsem)
          for ci in range(CCHUNK)]
    wd = [pltpu.make_async_copy(
              rows_v.at[ci % NBUF],
              out_hbm.at[pl.ds(base + ci * GCHUNK, GCHUNK), :], wsem)
          for ci in range(CCHUNK)]
    for ci in range(CCHUNK):
        if ci >= NBUF:
            wd[ci - NBUF].wait()
        gd[ci].start()
        if ci >= 1:
            gd[ci - 1].wait()
            wd[ci - 1].start()
    last = CCHUNK - 1
    gd[last].wait()
    wd[last].start()
    for ci in range(max(0, CCHUNK - NBUF + 1), CCHUNK):
        wd[ci].wait()


@functools.cache
def _combine():
    return pl.kernel(
        _combine_body,
        out_type=jax.ShapeDtypeStruct((N, D), jnp.float32),
        mesh=plsc.VectorSubcoreMesh(core_axis_name="c", subcore_axis_name="s"),
        scratch_types=[
            pltpu.VMEM((TOK_PER_W,), jnp.int32),
            pltpu.VMEM((NBUF, GCHUNK, D), jnp.float32),
            pltpu.SemaphoreType.DMA,
            pltpu.SemaphoreType.DMA,
        ],
    )


def kernel(hidden_states, gate_w, w1, w2, w3):
    x = hidden_states.reshape(N, D)
    logits, idx = _routing(x, gate_w)
    buf = _dispatch()(idx, x).reshape(E * CAP, D)
    ob = _ffn(buf, w1, w3, w2)
    out = _combine()(idx, ob)
    return out.reshape(B, S, D), logits


# speedup vs baseline: 1.8315x; 1.0017x over previous
import functools

import jax
import jax.numpy as jnp
from jax import lax
from jax.experimental import pallas as pl
from jax.experimental.pallas import tpu as pltpu
from jax.experimental.pallas import tpu_sc as plsc

B, S, D = 2, 2048, 1024
E, FF = 64, 1024
N = B * S
CAP = 2 * ((N + E - 1) // E)
SENTINEL = E * CAP

TB = 512
NC, NS = 2, 16
NW = NC * NS
SLOTS_PER_W = E * CAP // NW
TOK_PER_W = N // NW
GCHUNK = 32
NBUF = 3


def _routing_body(x_ref, gw_ref, logits_ref, idx_ref, base_ref):
    t = pl.program_id(0)

    @pl.when(t == 0)
    def _():
        base_ref[...] = jnp.zeros_like(base_ref)

    x = x_ref[...]
    logits = lax.dot_general(x, gw_ref[...], (((1,), (1,)), ((), ())),
                             preferred_element_type=jnp.float32)
    logits_ref[...] = logits
    m = jnp.max(logits, axis=1, keepdims=True)
    eidx = lax.broadcasted_iota(jnp.int32, (TB, E), 1)
    sel = jnp.min(jnp.where(logits == m, eidx, E), axis=1)
    oh = (eidx == sel[:, None]).astype(jnp.float32)
    r = lax.broadcasted_iota(jnp.int32, (TB, TB), 0)
    c = lax.broadcasted_iota(jnp.int32, (TB, TB), 1)
    ltri = (c <= r).astype(jnp.float32)
    csum = lax.dot_general(ltri, oh, (((1,), (0,)), ((), ())),
                           preferred_element_type=jnp.float32)
    csum = csum + base_ref[...]
    base_ref[...] = csum[TB - 1:TB, :]
    pos = jnp.sum(csum * oh, axis=1).astype(jnp.int32) - 1
    slot = sel * CAP + pos
    idx_ref[...] = jnp.where(pos < CAP, slot, SENTINEL)


def _routing(x, gate_w):
    return pl.pallas_call(
        _routing_body,
        grid=(N // TB,),
        in_specs=[
            pl.BlockSpec((TB, D), lambda t: (t, 0)),
            pl.BlockSpec((E, D), lambda t: (0, 0)),
        ],
        out_specs=[
            pl.BlockSpec((TB, E), lambda t: (t, 0)),
            pl.BlockSpec((TB,), lambda t: (t,)),
        ],
        out_shape=[
            jax.ShapeDtypeStruct((N, E), jnp.float32),
            jax.ShapeDtypeStruct((N,), jnp.int32),
        ],
        scratch_shapes=[pltpu.VMEM((1, E), jnp.float32)],
    )(x, gate_w)


NCHUNK = SLOTS_PER_W // GCHUNK


def _dispatch_body(idx_hbm, x_hbm, buf_hbm, idx_v, tid_v, rows_v,
                   gsem, wsem):
    wid = lax.axis_index("s") * NC + lax.axis_index("c")
    base_slot = pl.multiple_of(wid * SLOTS_PER_W, SLOTS_PER_W)
    pltpu.sync_copy(idx_hbm, idx_v)
    iota16 = lax.iota(jnp.int32, 16)
    neg16 = jnp.full((16,), -1, jnp.int32)
    for ci in range(NCHUNK):
        for i in range(GCHUNK // 16):
            tid_v[ci, pl.ds(i * 16, 16)] = neg16

    def body(j, carry):
        sl = idx_v[pl.ds(j * 16, 16)]
        rel = sl - base_slot
        msk = (rel >= 0) & (rel < SLOTS_PER_W)
        relc = jnp.clip(rel, 0, SLOTS_PER_W - 1)
        tok = lax.iota(jnp.int32, 16) + j * 16
        plsc.store_scatter(tid_v, [relc // GCHUNK, relc % GCHUNK], tok,
                           mask=msk)
        return carry

    lax.fori_loop(0, N // 16, body, 0)
    filled = []
    for ci in range(NCHUNK):
        filled.append(jnp.max(tid_v[ci, pl.ds(0, 16)]) >= 0)
        for i in range(GCHUNK // 16):
            v = tid_v[ci, pl.ds(i * 16, 16)]
            fillv = iota16 + ((base_slot + ci * GCHUNK + i * 16) & (N - 1))
            tid_v[ci, pl.ds(i * 16, 16)] = jnp.where(v < 0, fillv, v)
    gd = [pltpu.make_async_copy(x_hbm.at[tid_v.at[ci]],
                                rows_v.at[ci % NBUF], gsem)
          for ci in range(NCHUNK)]
    wd = [pltpu.make_async_copy(rows_v.at[ci % NBUF],
                                buf_hbm.at[wid * NCHUNK + ci], wsem)
          for ci in range(NCHUNK)]

    def when_do(cond, *fns):
        @pl.when(cond)
        def _():
            for f in fns:
                f()

    for ci in range(NCHUNK):
        if ci >= NBUF:
            when_do(filled[ci - NBUF], wd[ci - NBUF].wait)
        when_do(filled[ci], gd[ci].start)
        if ci >= 1:
            when_do(filled[ci - 1], gd[ci - 1].wait, wd[ci - 1].start)
    last = NCHUNK - 1
    when_do(filled[last], gd[last].wait, wd[last].start)
    for ci in range(max(0, NCHUNK - NBUF), NCHUNK - 1):
        when_do(filled[ci], wd[ci].wait)
    when_do(filled[last], wd[last].wait)


@functools.cache
def _dispatch():
    return pl.kernel(
        _dispatch_body,
        out_type=jax.ShapeDtypeStruct((E * CAP // GCHUNK, GCHUNK, D),
                                      jnp.float32),
        mesh=plsc.VectorSubcoreMesh(core_axis_name="c", subcore_axis_name="s"),
        scratch_types=[
            pltpu.VMEM((N,), jnp.int32),
            pltpu.VMEM((NCHUNK, GCHUNK), jnp.int32),
            pltpu.VMEM((NBUF, GCHUNK, D), jnp.float32),
            pltpu.SemaphoreType.DMA,
            pltpu.SemaphoreType.DMA,
        ],
        compiler_params=pltpu.CompilerParams(needs_layout_passes=False),
    )


def _ffn_body(buf_ref, w1_ref, w3_ref, w2_ref, ob_ref):
    e = pl.program_id(0)

    @pl.when(e < E)
    def _():
        x = buf_ref[...]
        a = jnp.dot(x, w1_ref[0], preferred_element_type=jnp.float32)
        b = jnp.dot(x, w3_ref[0], preferred_element_type=jnp.float32)
        g = a * (1.0 / (1.0 + jnp.exp(-a))) * b
        ob_ref[...] = jnp.dot(g, w2_ref[0], preferred_element_type=jnp.float32)

    @pl.when(e == E)
    def _():
        ob_ref[...] = jnp.zeros_like(ob_ref)


def _ffn(buf, w1, w3, w2):
    wmap = lambda e: (jnp.minimum(e, E - 1), 0, 0)
    return pl.pallas_call(
        _ffn_body,
        grid=(E + 1,),
        in_specs=[
            pl.BlockSpec((CAP, D), lambda e: (jnp.minimum(e, E - 1), 0)),
            pl.BlockSpec((1, D, FF), wmap),
            pl.BlockSpec((1, D, FF), wmap),
            pl.BlockSpec((1, FF, D), wmap),
        ],
        out_specs=pl.BlockSpec((CAP, D), lambda e: (e, 0)),
        out_shape=jax.ShapeDtypeStruct((E * CAP + CAP, D), jnp.float32),
    )(buf, w1, w3, w2)


CCHUNK = TOK_PER_W // GCHUNK


def _combine_body(idx_hbm, ob_hbm, out_hbm, idx_v, rows_v, gsem, wsem):
    wid = lax.axis_index("s") * NC + lax.axis_index("c")
    base = pl.multiple_of(wid * TOK_PER_W, TOK_PER_W)
    pltpu.sync_copy(idx_hbm.at[pl.ds(wid * CCHUNK, CCHUNK)], idx_v)
    gd = [pltpu.make_async_copy(
              ob_hbm.at[idx_v.at[ci]],
              rows_v.at[ci % NBUF], gsem)
          for ci in range(CCHUNK)]
    wd = [pltpu.make_async_copy(
              rows_v.at[ci % NBUF],
              out_hbm.at[pl.ds(base + ci * GCHUNK, GCHUNK), :], wsem)
          for ci in range(CCHUNK)]
    for ci in range(CCHUNK):
        if ci >= NBUF:
            wd[ci - NBUF].wait()
        gd[ci].start()
        if ci >= 1:
            gd[ci - 1].wait()
            wd[ci - 1].start()
    last = CCHUNK - 1
    gd[last].wait()
    wd[last].start()
    for ci in range(max(0, CCHUNK - NBUF), CCHUNK):
        wd[ci].wait()


@functools.cache
def _combine():
    return pl.kernel(
        _combine_body,
        out_type=jax.ShapeDtypeStruct((N, D), jnp.float32),
        mesh=plsc.VectorSubcoreMesh(core_axis_name="c", subcore_axis_name="s"),
        scratch_types=[
            pltpu.VMEM((CCHUNK, GCHUNK), jnp.int32),
            pltpu.VMEM((NBUF, GCHUNK, D), jnp.float32),
            pltpu.SemaphoreType.DMA,
            pltpu.SemaphoreType.DMA,
        ],
    )


def kernel(hidden_states, gate_w, w1, w2, w3):
    x = hidden_states.reshape(N, D)
    logits, idx = _routing(x, gate_w)
    buf = _dispatch()(idx, x).reshape(E * CAP, D)
    ob = _ffn(buf, w1, w3, w2)
    out = _combine()(idx.reshape(N // GCHUNK, GCHUNK), ob)
    return out.reshape(B, S, D), logits
